# baseline (speedup 1.0000x reference)
import functools

import jax
import jax.numpy as jnp
from jax import lax
from jax.experimental import pallas as pl
from jax.experimental.pallas import tpu as pltpu
from jax.experimental.pallas import tpu_sc as plsc

_M = 8
_K = 256
_D = 256
_DSUB = _D // _M
_BLK = 4096

_SC_NC = 2
_SC_NS = 16
_SC_NW = _SC_NC * _SC_NS


def _assign_body(x1_ref, cen_ref, lab_ref, maxs_ref):
    x1 = x1_ref[0]
    cm = cen_ref[0]
    csq = jnp.sum(cm * cm, axis=1, keepdims=True)
    xsq = jnp.sum(x1 * x1, axis=0, keepdims=True)
    scores = lax.dot_general(
        cm, x1, (((1,), (0,)), ((), ())),
        preferred_element_type=jnp.float32)
    adj2 = (csq - 2.0 * scores) + xsq
    dist = -jnp.sqrt(adj2)
    maxv = jnp.max(dist, axis=0, keepdims=True)
    kiota = lax.broadcasted_iota(jnp.int32, dist.shape, 0)
    lab = jnp.min(jnp.where(dist == maxv, kiota, _K),
                  axis=0, keepdims=True)
    lab_ref[...] = lab[None]
    maxs_ref[...] = maxv[None]


def _assign(X1, center, off, bh):
    nbh = bh // _BLK
    return pl.pallas_call(
        _assign_body,
        grid=(_M, nbh),
        in_specs=[
            pl.BlockSpec((1, _DSUB, _BLK), lambda m, i: (m, 0, i + off)),
            pl.BlockSpec((1, _K, _DSUB), lambda m, i: (m, 0, 0)),
        ],
        out_specs=[
            pl.BlockSpec((1, 1, _BLK), lambda m, i: (m, 0, i)),
            pl.BlockSpec((1, 1, _BLK), lambda m, i: (m, 0, i)),
        ],
        out_shape=[
            jax.ShapeDtypeStruct((_M, 1, bh), jnp.int32),
            jax.ShapeDtypeStruct((_M, 1, bh), jnp.float32),
        ],
    )(X1, center)


def _sc_gather(table, idx):
    n = idx.shape[0]
    bpw = n // _SC_NW

    @functools.partial(
        pl.kernel,
        mesh=plsc.VectorSubcoreMesh(core_axis_name="c", subcore_axis_name="s"),
        out_type=jax.ShapeDtypeStruct((n, _DSUB), jnp.float32),
        scratch_types=[
            pltpu.VMEM((bpw,), jnp.int32),
            pltpu.VMEM((bpw, _DSUB), jnp.float32),
            pltpu.SemaphoreType.DMA,
        ],
        compiler_params=pltpu.CompilerParams(use_tc_tiling_on_sc=False),
    )
    def gk(table_hbm, idx_hbm, out_hbm, idx_v, rows_v, sem):
        wid = lax.axis_index("s") * _SC_NC + lax.axis_index("c")
        base = wid * bpw
        pltpu.sync_copy(idx_hbm.at[pl.ds(base, bpw)], idx_v)
        pltpu.async_copy(table_hbm.at[idx_v], rows_v, sem).wait()
        pltpu.sync_copy(rows_v, out_hbm.at[pl.ds(base, bpw)])

    return gk(table, idx)


_RS = 512


def _repack_body(rows_ref, out_ref):
    out_ref[...] = rows_ref[...].reshape(_RS, _M, _DSUB)


def _repack(rows, B):
    nr = B // _RS
    return pl.pallas_call(
        _repack_body,
        grid=(nr,),
        in_specs=[pl.BlockSpec((_RS * _M, _DSUB), lambda i: (i, 0))],
        out_specs=pl.BlockSpec((_RS, _M, _DSUB), lambda i: (i, 0, 0)),
        out_shape=jax.ShapeDtypeStruct((B, _M, _DSUB), jnp.float32),
    )(rows)


def kernel(X, center):
    B = X.shape[0]
    X1 = jnp.transpose(X.reshape(B, _M, _DSUB), (1, 2, 0))
    lab3, maxs3 = _assign(X1, center, 0, B)
    lab_bm = jnp.swapaxes(lab3[:, 0, :], 0, 1)
    idx = (lab_bm + jnp.arange(_M, dtype=jnp.int32) * _K).reshape(B * _M)
    rows = _sc_gather(center.reshape(_M * _K, _DSUB), idx)
    X_r_out = _repack(rows, B)
    X_r_m = rows.reshape(B, _D)
    X_p = X.reshape(B, _M, _DSUB)
    label = lab_bm[..., None]
    loss = jnp.sum(maxs3 * maxs3) * jnp.float32(2.0 / (B * _D))
    return (X_r_out, X_p, X_r_m, X, center, label, loss)


# speedup vs baseline: 1.1462x; 1.1462x over previous
import functools

import jax
import jax.numpy as jnp
from jax import lax
from jax.experimental import pallas as pl
from jax.experimental.pallas import tpu as pltpu
from jax.experimental.pallas import tpu_sc as plsc

_M = 8
_K = 256
_D = 256
_DSUB = _D // _M
_BLK = 4096

_SC_NC = 2
_SC_NS = 16
_SC_NW = _SC_NC * _SC_NS


def _assign_body(x1_ref, cen_ref, lab_ref, maxs_ref):
    x1 = x1_ref[0]
    cm = cen_ref[0]
    csq = jnp.sum(cm * cm, axis=1, keepdims=True)
    xsq = jnp.sum(x1 * x1, axis=0, keepdims=True)
    scores = lax.dot_general(
        cm, x1, (((1,), (0,)), ((), ())),
        preferred_element_type=jnp.float32)
    adj2 = (csq - 2.0 * scores) + xsq
    dist = -jnp.sqrt(adj2)
    maxv = jnp.max(dist, axis=0, keepdims=True)
    kiota = lax.broadcasted_iota(jnp.int32, dist.shape, 0)
    lab = jnp.min(jnp.where(dist == maxv, kiota, _K),
                  axis=0, keepdims=True)
    lab_ref[...] = lab[None]
    maxs_ref[...] = maxv[None]


def _assign(X1, center, off, bh):
    nbh = bh // _BLK
    return pl.pallas_call(
        _assign_body,
        grid=(_M, nbh),
        in_specs=[
            pl.BlockSpec((1, _DSUB, _BLK), lambda m, i: (m, 0, i + off)),
            pl.BlockSpec((1, _K, _DSUB), lambda m, i: (m, 0, 0)),
        ],
        out_specs=[
            pl.BlockSpec((1, 1, _BLK), lambda m, i: (m, 0, i)),
            pl.BlockSpec((1, 1, _BLK), lambda m, i: (m, 0, i)),
        ],
        out_shape=[
            jax.ShapeDtypeStruct((_M, 1, bh), jnp.int32),
            jax.ShapeDtypeStruct((_M, 1, bh), jnp.float32),
        ],
    )(X1, center)


def _sc_gather(table, idx):
    n = idx.shape[0]
    bpw = n // _SC_NW

    @functools.partial(
        pl.kernel,
        mesh=plsc.VectorSubcoreMesh(core_axis_name="c", subcore_axis_name="s"),
        out_type=jax.ShapeDtypeStruct((n, _DSUB), jnp.float32),
        scratch_types=[
            pltpu.VMEM((bpw,), jnp.int32),
            pltpu.VMEM((bpw, _DSUB), jnp.float32),
            pltpu.SemaphoreType.DMA,
        ],
        compiler_params=pltpu.CompilerParams(use_tc_tiling_on_sc=False),
    )
    def gk(table_hbm, idx_hbm, out_hbm, idx_v, rows_v, sem):
        wid = lax.axis_index("s") * _SC_NC + lax.axis_index("c")
        base = wid * bpw
        pltpu.sync_copy(idx_hbm.at[pl.ds(base, bpw)], idx_v)
        pltpu.async_copy(table_hbm.at[idx_v], rows_v, sem).wait()
        pltpu.sync_copy(rows_v, out_hbm.at[pl.ds(base, bpw)])

    return gk(table, idx)


def kernel(X, center):
    B = X.shape[0]
    X1 = jnp.transpose(X.reshape(B, _M, _DSUB), (1, 2, 0))
    lab3, maxs3 = _assign(X1, center, 0, B)
    lab_bm = jnp.swapaxes(lab3[:, 0, :], 0, 1)
    idx = (lab_bm + jnp.arange(_M, dtype=jnp.int32) * _K).reshape(B * _M)
    rows = _sc_gather(center.reshape(_M * _K, _DSUB), idx)
    X_r_out = rows.reshape(B, _M, _DSUB)
    X_r_m = rows.reshape(B, _D)
    X_p = X.reshape(B, _M, _DSUB)
    label = lab_bm[..., None]
    loss = jnp.sum(maxs3 * maxs3) * jnp.float32(2.0 / (B * _D))
    return (X_r_out, X_p, X_r_m, X, center, label, loss)


# speedup vs baseline: 1.1528x; 1.0058x over previous
import functools

import jax
import jax.numpy as jnp
from jax import lax
from jax.experimental import pallas as pl
from jax.experimental.pallas import tpu as pltpu
from jax.experimental.pallas import tpu_sc as plsc

_M = 8
_K = 256
_D = 256
_DSUB = _D // _M
_BLK = 4096

_SC_NC = 2
_SC_NS = 16
_SC_NW = _SC_NC * _SC_NS


def _assign_body(x1_ref, cen_ref, lab_ref, loss_ref):
    x1 = x1_ref[0]
    cm = cen_ref[0]
    csq = jnp.sum(cm * cm, axis=1, keepdims=True)
    xsq = jnp.sum(x1 * x1, axis=0, keepdims=True)
    scores = lax.dot_general(
        cm, x1, (((1,), (0,)), ((), ())),
        preferred_element_type=jnp.float32)
    adj2 = (csq - 2.0 * scores) + xsq
    dist = -jnp.sqrt(adj2)
    maxv = jnp.max(dist, axis=0, keepdims=True)
    kiota = lax.broadcasted_iota(jnp.int32, dist.shape, 0)
    lab = jnp.min(jnp.where(dist == maxv, kiota, _K),
                  axis=0, keepdims=True)
    lab_ref[...] = lab[None]
    partial = jnp.sum(maxv * maxv)
    loss_ref[...] = jnp.full((1, 1, 8, 128), partial, jnp.float32)


def _assign(X1, center, off, bh):
    nbh = bh // _BLK
    return pl.pallas_call(
        _assign_body,
        grid=(_M, nbh),
        in_specs=[
            pl.BlockSpec((1, _DSUB, _BLK), lambda m, i: (m, 0, i + off)),
            pl.BlockSpec((1, _K, _DSUB), lambda m, i: (m, 0, 0)),
        ],
        out_specs=[
            pl.BlockSpec((1, 1, _BLK), lambda m, i: (m, 0, i)),
            pl.BlockSpec((1, 1, 8, 128), lambda m, i: (m, i, 0, 0)),
        ],
        out_shape=[
            jax.ShapeDtypeStruct((_M, 1, bh), jnp.int32),
            jax.ShapeDtypeStruct((_M, nbh, 8, 128), jnp.float32),
        ],
    )(X1, center)


def _sc_gather(table, idx):
    n = idx.shape[0]
    bpw = n // _SC_NW

    @functools.partial(
        pl.kernel,
        mesh=plsc.VectorSubcoreMesh(core_axis_name="c", subcore_axis_name="s"),
        out_type=jax.ShapeDtypeStruct((n, _DSUB), jnp.float32),
        scratch_types=[
            pltpu.VMEM((bpw,), jnp.int32),
            pltpu.VMEM((bpw, _DSUB), jnp.float32),
            pltpu.SemaphoreType.DMA,
        ],
        compiler_params=pltpu.CompilerParams(use_tc_tiling_on_sc=False),
    )
    def gk(table_hbm, idx_hbm, out_hbm, idx_v, rows_v, sem):
        wid = lax.axis_index("s") * _SC_NC + lax.axis_index("c")
        base = wid * bpw
        pltpu.sync_copy(idx_hbm.at[pl.ds(base, bpw)], idx_v)
        pltpu.async_copy(table_hbm.at[idx_v], rows_v, sem).wait()
        pltpu.sync_copy(rows_v, out_hbm.at[pl.ds(base, bpw)])

    return gk(table, idx)


def kernel(X, center):
    B = X.shape[0]
    X1 = jnp.transpose(X.reshape(B, _M, _DSUB), (1, 2, 0))
    lab3, lossp = _assign(X1, center, 0, B)
    lab_bm = jnp.swapaxes(lab3[:, 0, :], 0, 1)
    idx = (lab_bm + jnp.arange(_M, dtype=jnp.int32) * _K).reshape(B * _M)
    rows = _sc_gather(center.reshape(_M * _K, _DSUB), idx)
    X_r_out = rows.reshape(B, _M, _DSUB)
    X_r_m = rows.reshape(B, _D)
    X_p = X.reshape(B, _M, _DSUB)
    label = lab_bm[..., None]
    loss = jnp.sum(lossp[:, :, 0, 0]) * jnp.float32(2.0 / (B * _D))
    return (X_r_out, X_p, X_r_m, X, center, label, loss)
